# baseline (speedup 1.0000x reference)
import jax
import jax.numpy as jnp
import numpy as np
from jax import lax
from jax.experimental import pallas as pl
from jax.experimental.pallas import tpu as pltpu
from jax.experimental.pallas import tpu_sc as plsc

_N = 36
_PERM = tuple(int(v) for v in np.random.RandomState(0).permutation(_N))

_B = 4096
_D = 128
_NC = 2
_NS = 16
_WIN = 256
_HALF = _N // 2
_NA = 10
_NB = _HALF - _NA
_NBUF = 3


def _ring(n, nbuf, start_in, wait_in, start_out, wait_out):
    for b in range(min(nbuf, n)):
        start_in(b, b)
    for t in range(n):
        b = t % nbuf
        wait_in(t, b)
        start_out(t, b)
        r = t + 1
        if nbuf <= r < n:
            rb = r % nbuf
            wait_out(r - nbuf, rb)
            start_in(r, rb)
    for t in range(max(n - nbuf, 0), n):
        wait_out(t, t % nbuf)


def _run_a(x_hbm, out_hbm, bufs, semr, semw, b0, j0):
    def src(t):
        return x_hbm.at[_PERM[j0 + t], pl.ds(b0, _WIN), :]

    def dst(t):
        return out_hbm.at[j0 + t, pl.ds(b0, _WIN), :]

    def start_in(t, b):
        pltpu.async_copy(src(t), bufs[b], semr[b])

    def wait_in(t, b):
        pltpu.make_async_copy(src(t), bufs[b], semr[b]).wait()

    def start_out(t, b):
        pltpu.async_copy(bufs[b], dst(t), semw[b])

    def wait_out(t, b):
        pltpu.make_async_copy(bufs[b], dst(t), semw[b]).wait()

    _ring(_NA, 2, start_in, wait_in, start_out, wait_out)


_BH = _B // 2


def _run_b(x_hbm, out_hbm, sbuf, semr, semw, j0):
    tasks = []
    for k in range(_NB):
        j = j0 + _NA + k
        for h in range(2):
            tasks.append(
                (
                    x_hbm.at[_PERM[j], pl.ds(h * _BH, _BH), :],
                    out_hbm.at[j, pl.ds(h * _BH, _BH), :],
                    _BH,
                )
            )
    for k in range(_NA):
        j = j0 + k
        tasks.append(
            (
                x_hbm.at[_PERM[j], pl.ds(0, _WIN), :],
                out_hbm.at[j, pl.ds(0, _WIN), :],
                _WIN,
            )
        )

    def buf(b, sz):
        return sbuf.at[b, pl.ds(0, sz), :]

    def start_in(t, b):
        src, _, sz = tasks[t]
        pltpu.async_copy(src, buf(b, sz), semr[b])

    def wait_in(t, b):
        src, _, sz = tasks[t]
        pltpu.make_async_copy(src, buf(b, sz), semr[b]).wait()

    def start_out(t, b):
        _, dst, sz = tasks[t]
        pltpu.async_copy(buf(b, sz), dst, semw[b])

    def wait_out(t, b):
        _, dst, sz = tasks[t]
        pltpu.make_async_copy(buf(b, sz), dst, semw[b]).wait()

    _ring(len(tasks), _NBUF, start_in, wait_in, start_out, wait_out)


def _body(x_hbm, out_hbm, buf0, buf1, sbuf,
          semra0, semra1, semwa0, semwa1,
          semrb0, semrb1, semrb2, semwb0, semwb1, semwb2):
    c = lax.axis_index("c")
    s = lax.axis_index("s")
    b0 = s * _WIN
    bufs = (buf0, buf1)
    semra = (semra0, semra1)
    semwa = (semwa0, semwa1)
    semrb = (semrb0, semrb1, semrb2)
    semwb = (semwb0, semwb1, semwb2)

    @pl.when(c == 0)
    def _():
        @pl.when(s == 0)
        def _():
            _run_b(x_hbm, out_hbm, sbuf, semrb, semwb, 0)

        @pl.when(s != 0)
        def _():
            _run_a(x_hbm, out_hbm, bufs, semra, semwa, b0, 0)

    @pl.when(c == 1)
    def _():
        @pl.when(s == 0)
        def _():
            _run_b(x_hbm, out_hbm, sbuf, semrb, semwb, _HALF)

        @pl.when(s != 0)
        def _():
            _run_a(x_hbm, out_hbm, bufs, semra, semwa, b0, _HALF)


@jax.jit
def kernel(x):
    xt = jnp.transpose(x, (1, 0, 2))
    mesh = plsc.VectorSubcoreMesh(core_axis_name="c", subcore_axis_name="s")
    out_t = pl.kernel(
        _body,
        out_type=jax.ShapeDtypeStruct((_N, _B, _D), x.dtype),
        mesh=mesh,
        scratch_types=[
            pltpu.VMEM((_WIN, _D), jnp.float32),
            pltpu.VMEM((_WIN, _D), jnp.float32),
            pltpu.VMEM_SHARED((_NBUF, _B // 2, _D), jnp.float32),
            pltpu.SemaphoreType.DMA,
            pltpu.SemaphoreType.DMA,
            pltpu.SemaphoreType.DMA,
            pltpu.SemaphoreType.DMA,
            pltpu.SemaphoreType.DMA,
            pltpu.SemaphoreType.DMA,
            pltpu.SemaphoreType.DMA,
            pltpu.SemaphoreType.DMA,
            pltpu.SemaphoreType.DMA,
            pltpu.SemaphoreType.DMA,
        ],
    )(xt)
    return jnp.transpose(out_t, (1, 0, 2))


# speedup vs baseline: 1.0966x; 1.0966x over previous
import jax
import jax.numpy as jnp
import numpy as np
from jax import lax
from jax.experimental import pallas as pl
from jax.experimental.pallas import tpu as pltpu
from jax.experimental.pallas import tpu_sc as plsc

_N = 36
_PERM = tuple(int(v) for v in np.random.RandomState(0).permutation(_N))

_B = 4096
_D = 128
_NC = 2
_NS = 16
_WIN = 256
_HALF = _N // 2
_NBUF = 3


def _run(x_hbm, out_hbm, bufs, semr, semw, b0, j0):
    def start_in(j, b):
        pltpu.async_copy(
            x_hbm.at[_PERM[j0 + j], pl.ds(b0, _WIN), :], bufs[b], semr[b]
        )

    def wait_in(j, b):
        pltpu.make_async_copy(
            x_hbm.at[_PERM[j0 + j], pl.ds(b0, _WIN), :], bufs[b], semr[b]
        ).wait()

    def start_out(j, b):
        pltpu.async_copy(
            bufs[b], out_hbm.at[j0 + j, pl.ds(b0, _WIN), :], semw[b]
        )

    def wait_out(j, b):
        pltpu.make_async_copy(
            bufs[b], out_hbm.at[j0 + j, pl.ds(b0, _WIN), :], semw[b]
        ).wait()

    for b in range(_NBUF):
        start_in(b, b)

    for t in range(_HALF):
        b = t % _NBUF
        wait_in(t, b)
        start_out(t, b)
        r = t + 1
        if _NBUF <= r < _HALF:
            rb = r % _NBUF
            wait_out(r - _NBUF, rb)
            start_in(r, rb)

    for t in range(_HALF - _NBUF, _HALF):
        wait_out(t, t % _NBUF)


def _body(x_hbm, out_hbm, buf0, buf1, buf2, semr0, semr1, semr2,
          semw0, semw1, semw2):
    c = lax.axis_index("c")
    s = lax.axis_index("s")
    b0 = s * _WIN
    bufs = (buf0, buf1, buf2)
    semr = (semr0, semr1, semr2)
    semw = (semw0, semw1, semw2)

    @pl.when(c == 0)
    def _():
        _run(x_hbm, out_hbm, bufs, semr, semw, b0, 0)

    @pl.when(c == 1)
    def _():
        _run(x_hbm, out_hbm, bufs, semr, semw, b0, _HALF)


@jax.jit
def kernel(x):
    xt = jnp.transpose(x, (1, 0, 2))
    mesh = plsc.VectorSubcoreMesh(core_axis_name="c", subcore_axis_name="s")
    out_t = pl.kernel(
        _body,
        out_type=jax.ShapeDtypeStruct((_N, _B, _D), x.dtype),
        mesh=mesh,
        scratch_types=[
            pltpu.VMEM((_WIN, _D), jnp.float32),
            pltpu.VMEM((_WIN, _D), jnp.float32),
            pltpu.VMEM((_WIN, _D), jnp.float32),
            pltpu.SemaphoreType.DMA,
            pltpu.SemaphoreType.DMA,
            pltpu.SemaphoreType.DMA,
            pltpu.SemaphoreType.DMA,
            pltpu.SemaphoreType.DMA,
            pltpu.SemaphoreType.DMA,
        ],
    )(xt)
    return jnp.transpose(out_t, (1, 0, 2))
